# Initial kernel scaffold; baseline (speedup 1.0000x reference)
import jax
import jax.numpy as jnp
from jax.experimental import pallas as pl


def kernel(adj, x, eps1, W11, b11, g11, be11, W12, b12, g12, be12, eps2, W21, b21, g21, be21, W22, b22, g22, be22):
    raise NotImplementedError("write your pallas kernel here")


# speedup vs baseline: 4.6412x; 4.6412x over previous
import functools

import jax
import jax.numpy as jnp
from jax import lax
from jax.experimental import pallas as pl
from jax.experimental.pallas import tpu as pltpu
from jax.experimental.pallas import tpu_sc as plsc

N = 10000
E = 320000
D = 128

_NC = 2
_NS = 16
_CH = 80
_EPT = E // (_NC * _NS)
_NCHUNK = _EPT // _CH
_RPB = 624
_RTAIL = N - _NS * _RPB


def _sc_agg_body(x_hbm, src_hbm, dst_hbm, zeros_hbm, out_hbm,
                 agg_sh, src_v, dst_v, rows_v, sem):
    cid = lax.axis_index("c")
    sid = lax.axis_index("s")

    if True:
        row0 = sid * _RPB
        pltpu.sync_copy(zeros_hbm.at[pl.ds(row0, _RPB)],
                        agg_sh.at[pl.ds(row0, _RPB)])

        @pl.when(sid == _NS - 1)
        def _():
            pltpu.sync_copy(zeros_hbm.at[pl.ds(_NS * _RPB, _RTAIL)],
                            agg_sh.at[pl.ds(_NS * _RPB, _RTAIL)])

        plsc.subcore_barrier()

        ebase = cid * (E // _NC) + sid * _EPT

        def step(k, _):
            base = ebase + k * _CH
            pltpu.sync_copy(src_hbm.at[pl.ds(base, _CH)], src_v)
            pltpu.sync_copy(dst_hbm.at[pl.ds(base, _CH)], dst_v)
            pltpu.async_copy(x_hbm.at[src_v], rows_v, sem).wait()
            pltpu.sync_copy(rows_v, agg_sh.at[dst_v], add=True)
            return _

        lax.fori_loop(0, _NCHUNK, step, 0)
        plsc.subcore_barrier()

        pltpu.sync_copy(agg_sh.at[pl.ds(row0, _RPB)],
                        out_hbm.at[cid, pl.ds(row0, _RPB)])

        @pl.when(sid == _NS - 1)
        def _():
            pltpu.sync_copy(agg_sh.at[pl.ds(_NS * _RPB, _RTAIL)],
                            out_hbm.at[cid, pl.ds(_NS * _RPB, _RTAIL)])


@functools.partial(jax.jit, static_argnames=())
def _sc_agg(x, src, dst, zeros):
    mesh = plsc.VectorSubcoreMesh(core_axis_name="c", subcore_axis_name="s")
    return pl.kernel(
        _sc_agg_body,
        out_type=jax.ShapeDtypeStruct((_NC, N, D), jnp.float32),
        mesh=mesh,
        scratch_types=[
            pltpu.VMEM_SHARED((N, D), jnp.float32),
            pltpu.VMEM((_CH,), jnp.int32),
            pltpu.VMEM((_CH,), jnp.int32),
            pltpu.VMEM((_CH, D), jnp.float32),
            pltpu.SemaphoreType.DMA,
        ],
    )(x, src, dst, zeros)


def _tc_mlp_body(eps_ref, x_ref, p_ref, W1_ref, b1_ref, g1_ref, be1_ref,
                 W2_ref, b2_ref, g2_ref, be2_ref, out_ref):
    eps = eps_ref[0, 0]
    h = x_ref[...] * (1.0 + eps) + p_ref[0] + p_ref[1]
    h = jnp.dot(h, W1_ref[...], preferred_element_type=jnp.float32) + b1_ref[...]
    mu = jnp.mean(h, axis=0, keepdims=True)
    var = jnp.mean((h - mu) * (h - mu), axis=0, keepdims=True)
    h = (h - mu) * lax.rsqrt(var + 1e-5) * g1_ref[...] + be1_ref[...]
    h = jnp.maximum(h, 0.0)
    h = jnp.dot(h, W2_ref[...], preferred_element_type=jnp.float32) + b2_ref[...]
    mu = jnp.mean(h, axis=0, keepdims=True)
    var = jnp.mean((h - mu) * (h - mu), axis=0, keepdims=True)
    h = (h - mu) * lax.rsqrt(var + 1e-5) * g2_ref[...] + be2_ref[...]
    out_ref[...] = jnp.maximum(h, 0.0)


def _tc_mlp(eps, x, parts, W1, b1, g1, be1, W2, b2, g2, be2):
    smem = pl.BlockSpec(memory_space=pltpu.SMEM)
    vmem = pl.BlockSpec(memory_space=pltpu.VMEM)
    return pl.pallas_call(
        _tc_mlp_body,
        out_shape=jax.ShapeDtypeStruct((N, D), jnp.float32),
        in_specs=[smem] + [vmem] * 10,
        out_specs=vmem,
    )(eps.reshape(1, 1), x, parts,
      W1, b1.reshape(1, D), g1.reshape(1, D), be1.reshape(1, D),
      W2, b2.reshape(1, D), g2.reshape(1, D), be2.reshape(1, D))


def kernel(adj, x, eps1, W11, b11, g11, be11, W12, b12, g12, be12,
           eps2, W21, b21, g21, be21, W22, b22, g22, be22):
    src = adj[0]
    dst = adj[1]
    zeros = jnp.zeros((N, D), jnp.float32)
    p1 = _sc_agg(x, src, dst, zeros)
    h = _tc_mlp(eps1, x, p1, W11, b11, g11, be11, W12, b12, g12, be12)
    p2 = _sc_agg(h, src, dst, zeros)
    h = _tc_mlp(eps2, h, p2, W21, b21, g21, be21, W22, b22, g22, be22)
    return h


# speedup vs baseline: 9.1204x; 1.9651x over previous
import functools

import jax
import jax.numpy as jnp
from jax import lax
from jax.experimental import pallas as pl
from jax.experimental.pallas import tpu as pltpu
from jax.experimental.pallas import tpu_sc as plsc

N = 10000
E = 320000
D = 128
_DH = D // 2

_NC = 2
_NS = 16
_CH = 80
_EPT = E // _NS
_NCHUNK = _EPT // _CH
_NBUF = 5
_NG = _NCHUNK // _NBUF
_RPB = 624
_RTAIL = N - _NS * _RPB


def _sc_agg_body(xc_hbm, src_hbm, dst_hbm, zeros_hbm, out_hbm, agg_sh, *scr):
    cid = lax.axis_index("c")
    sid = lax.axis_index("s")
    ebase = sid * _EPT

    k = 0

    def take(n):
        nonlocal k
        out, k = scr[k:k + n], k + n
        return [list(out[s * _NBUF:(s + 1) * _NBUF]) for s in range(2)]

    srcv = take(2 * _NBUF)
    dstv = take(2 * _NBUF)
    rows = take(2 * _NBUF)
    isem = take(2 * _NBUF)
    dsem = take(2 * _NBUF)

    def start_idx(g, s):
        for b in range(_NBUF):
            base = ebase + (g * _NBUF + b) * _CH
            pltpu.async_copy(src_hbm.at[pl.ds(base, _CH)], srcv[s][b], isem[s][b])
            pltpu.async_copy(dst_hbm.at[pl.ds(base, _CH)], dstv[s][b], isem[s][b])

    def wait_idx(s):
        for b in range(_NBUF):
            pltpu.make_async_copy(src_hbm.at[pl.ds(0, _CH)], srcv[s][b], isem[s][b]).wait()
            pltpu.make_async_copy(dst_hbm.at[pl.ds(0, _CH)], dstv[s][b], isem[s][b]).wait()

    def start_gathers(s):
        for b in range(_NBUF):
            pltpu.async_copy(xc_hbm.at[cid].at[srcv[s][b]], rows[s][b], dsem[s][b])

    def wait_gather(s, b):
        pltpu.make_async_copy(xc_hbm.at[cid].at[srcv[s][b]], rows[s][b], dsem[s][b]).wait()

    def start_scatter(s, b):
        pltpu.async_copy(rows[s][b], agg_sh.at[dstv[s][b]], dsem[s][b], add=True)

    def wait_scatter(s, b):
        pltpu.make_async_copy(rows[s][b], agg_sh.at[dstv[s][b]], dsem[s][b]).wait()

    row0 = sid * _RPB
    pltpu.sync_copy(zeros_hbm.at[pl.ds(row0, _RPB)], agg_sh.at[pl.ds(row0, _RPB)])

    @pl.when(sid == _NS - 1)
    def _():
        pltpu.sync_copy(zeros_hbm.at[pl.ds(_NS * _RPB, _RTAIL)],
                        agg_sh.at[pl.ds(_NS * _RPB, _RTAIL)])

    start_idx(0, 0)
    start_idx(1, 1)
    wait_idx(0)
    start_gathers(0)

    plsc.subcore_barrier()

    def group_body(g, s):
        o = 1 - s
        for b in range(_NBUF):
            wait_gather(s, b)
            start_scatter(s, b)

        @pl.when(g + 1 < _NG)
        def _():
            wait_idx(o)
            start_gathers(o)

        for b in range(_NBUF):
            wait_scatter(s, b)

        @pl.when(g + 2 < _NG)
        def _():
            start_idx(g + 2, s)

    def pair(gp, carry):
        g0 = gp * 2
        group_body(g0, 0)
        group_body(g0 + 1, 1)
        return carry

    lax.fori_loop(0, _NG // 2, pair, 0)

    plsc.subcore_barrier()

    pltpu.sync_copy(agg_sh.at[pl.ds(row0, _RPB)],
                    out_hbm.at[cid, pl.ds(row0, _RPB)])

    @pl.when(sid == _NS - 1)
    def _():
        pltpu.sync_copy(agg_sh.at[pl.ds(_NS * _RPB, _RTAIL)],
                        out_hbm.at[cid, pl.ds(_NS * _RPB, _RTAIL)])


@functools.partial(jax.jit, static_argnames=())
def _sc_agg(xc, src, dst, zeros):
    mesh = plsc.VectorSubcoreMesh(core_axis_name="c", subcore_axis_name="s")
    scratch = (
        [pltpu.VMEM_SHARED((N, _DH), jnp.float32)]
        + [pltpu.VMEM((_CH,), jnp.int32) for _ in range(2 * _NBUF)]
        + [pltpu.VMEM((_CH,), jnp.int32) for _ in range(2 * _NBUF)]
        + [pltpu.VMEM((_CH, _DH), jnp.float32) for _ in range(2 * _NBUF)]
        + [pltpu.SemaphoreType.DMA for _ in range(2 * _NBUF)]
        + [pltpu.SemaphoreType.DMA for _ in range(2 * _NBUF)]
    )
    return pl.kernel(
        _sc_agg_body,
        out_type=jax.ShapeDtypeStruct((_NC, N, _DH), jnp.float32),
        mesh=mesh,
        scratch_types=scratch,
        compiler_params=pltpu.CompilerParams(use_tc_tiling_on_sc=False),
    )(xc, src, dst, zeros)


def _tc_mlp_body(split_out, eps_ref, x_ref, p_ref, W1_ref, b1_ref, g1_ref,
                 be1_ref, W2_ref, b2_ref, g2_ref, be2_ref, out_ref, *split_ref):
    eps = eps_ref[0, 0]
    agg = jnp.concatenate([p_ref[0], p_ref[1]], axis=1)
    h = x_ref[...] * (1.0 + eps) + agg
    h = jnp.dot(h, W1_ref[...], preferred_element_type=jnp.float32) + b1_ref[...]
    mu = jnp.mean(h, axis=0, keepdims=True)
    var = jnp.mean((h - mu) * (h - mu), axis=0, keepdims=True)
    h = (h - mu) * lax.rsqrt(var + 1e-5) * g1_ref[...] + be1_ref[...]
    h = jnp.maximum(h, 0.0)
    h = jnp.dot(h, W2_ref[...], preferred_element_type=jnp.float32) + b2_ref[...]
    mu = jnp.mean(h, axis=0, keepdims=True)
    var = jnp.mean((h - mu) * (h - mu), axis=0, keepdims=True)
    h = (h - mu) * lax.rsqrt(var + 1e-5) * g2_ref[...] + be2_ref[...]
    h = jnp.maximum(h, 0.0)
    out_ref[...] = h
    if split_out:
        split_ref[0][0] = h[:, :_DH]
        split_ref[0][1] = h[:, _DH:]


def _tc_mlp(split_out, eps, x, parts, W1, b1, g1, be1, W2, b2, g2, be2):
    smem = pl.BlockSpec(memory_space=pltpu.SMEM)
    vmem = pl.BlockSpec(memory_space=pltpu.VMEM)
    out_shape = [jax.ShapeDtypeStruct((N, D), jnp.float32)]
    out_specs = [vmem]
    if split_out:
        out_shape.append(jax.ShapeDtypeStruct((_NC, N, _DH), jnp.float32))
        out_specs.append(vmem)
    return pl.pallas_call(
        functools.partial(_tc_mlp_body, split_out),
        out_shape=out_shape,
        in_specs=[smem] + [vmem] * 10,
        out_specs=out_specs,
    )(eps.reshape(1, 1), x, parts,
      W1, b1.reshape(1, D), g1.reshape(1, D), be1.reshape(1, D),
      W2, b2.reshape(1, D), g2.reshape(1, D), be2.reshape(1, D))


def kernel(adj, x, eps1, W11, b11, g11, be11, W12, b12, g12, be12,
           eps2, W21, b21, g21, be21, W22, b22, g22, be22):
    src = adj[0]
    dst = adj[1]
    zeros = jnp.zeros((N, _DH), jnp.float32)
    xc = x.reshape(N, _NC, _DH).transpose(1, 0, 2)
    p1 = _sc_agg(xc, src, dst, zeros)
    h, hc = _tc_mlp(True, eps1, x, p1, W11, b11, g11, be11, W12, b12, g12, be12)
    p2 = _sc_agg(hc, src, dst, zeros)
    (h,) = _tc_mlp(False, eps2, h, p2, W21, b21, g21, be21, W22, b22, g22, be22)
    return h
